# baseline (speedup 1.0000x reference)
import math

import jax
import jax.numpy as jnp
from jax import lax
from jax.experimental import pallas as pl
from jax.experimental.pallas import tpu as pltpu
from jax.experimental.pallas import tpu_sc as plsc

N = 10000
E = 320000
D = 128
H = 8
C = D // H
DK = 2 * D
WPAD = 16

NC, NS = 2, 16
NW = NC * NS
CHUNK = 128
NCHUNK = E // CHUNK
ITERS = (NCHUNK + NW - 1) // NW
NP = 10240
NHALF = NP // NC
SROWS = NHALF + 8
RPT_HALF = NHALF // NS
ITERS2 = (NCHUNK + NS - 1) // NS

RN = 1000
RN2 = 640
BE = 2000


def _node_tables_kernel(x_ref, wall_ref, ball_ref, q_ref, kv_ref, s_ref):
    t = jnp.dot(x_ref[...], wall_ref[...], preferred_element_type=jnp.float32) + ball_ref[...]
    q_ref[...] = t[:, :D]
    kv_ref[...] = t[:, D:D + DK].astype(jnp.bfloat16)
    s_ref[...] = x_ref[...] + t[:, D + DK:]


def _gather_body(q_hbm, kv_hbm, src_hbm, dst_hbm, gq_hbm, gkv_hbm,
                 idxs_v, idxd_v, qbuf, kvbuf, sem1, sem2):
    cid = lax.axis_index("c")
    sid = lax.axis_index("s")
    wid = sid * NC + cid

    def body(j, carry):
        c = j * NW + wid

        @pl.when(c < NCHUNK)
        def _():
            base = c * CHUNK
            pltpu.sync_copy(src_hbm.at[pl.ds(base, CHUNK)], idxs_v)
            pltpu.sync_copy(dst_hbm.at[pl.ds(base, CHUNK)], idxd_v)
            cp1 = pltpu.async_copy(kv_hbm.at[idxs_v], kvbuf, sem1)
            cp2 = pltpu.async_copy(q_hbm.at[idxd_v], qbuf, sem2)
            cp1.wait()
            cp2.wait()
            pltpu.sync_copy(kvbuf, gkv_hbm.at[pl.ds(base, CHUNK)])
            pltpu.sync_copy(qbuf, gq_hbm.at[pl.ds(base, CHUNK)])

        return carry

    lax.fori_loop(0, ITERS, body, 0)


def _edge_kernel(ea_ref, gq_ref, gkv_ref, wkv_ref, sel_ref, selt_ref,
                 wv_ref, w16_ref):
    ekv = jnp.dot(ea_ref[...], wkv_ref[...], preferred_element_type=jnp.float32)
    gkv = gkv_ref[...].astype(jnp.float32)
    k = gkv[:, :D] + ekv[:, :D]
    v = gkv[:, D:] + ekv[:, D:]
    alpha = jnp.dot(gq_ref[...] * k, sel_ref[...],
                    preferred_element_type=jnp.float32)
    w = jnp.exp(alpha)
    wfull = jnp.dot(w, selt_ref[...], preferred_element_type=jnp.float32)
    wv_ref[...] = v * wfull
    w16_ref[...] = jnp.concatenate([w, jnp.zeros_like(w)], axis=1)


def _scatter_body(wv_hbm, dst_hbm, zn_hbm, nump_hbm,
                  idx_v, wvbuf, num_sh, sem):
    cid = lax.axis_index("c")
    sid = lax.axis_index("s")
    wid = sid * NC + cid
    r0 = sid * (NP // NS)

    pltpu.sync_copy(zn_hbm.at[pl.ds(r0, NP // NS)],
                    num_sh.at[pl.ds(r0, NP // NS)])
    plsc.subcore_barrier()

    def body(j, carry):
        c = j * NW + wid

        @pl.when(c < NCHUNK)
        def _():
            base = c * CHUNK
            pltpu.sync_copy(dst_hbm.at[pl.ds(base, CHUNK)], idx_v)
            pltpu.sync_copy(wv_hbm.at[pl.ds(base, CHUNK)], wvbuf)
            pltpu.sync_copy(wvbuf, num_sh.at[idx_v], add=True)

        return carry

    lax.fori_loop(0, ITERS, body, 0)
    plsc.subcore_barrier()

    pltpu.sync_copy(num_sh.at[pl.ds(r0, NP // NS)],
                    nump_hbm.at[cid, pl.ds(r0, NP // NS)])


def _sums_body(w16_hbm, dst_hbm, zt_hbm, s_all_hbm, idx_v, wbuf, stab):
    cid = lax.axis_index("c")
    sid = lax.axis_index("s")
    wid = sid * NC + cid

    pltpu.sync_copy(zt_hbm, stab)

    def body(j, carry):
        c = j * NW + wid

        @pl.when(c < NCHUNK)
        def _():
            base = c * CHUNK
            pltpu.sync_copy(dst_hbm.at[pl.ds(base, CHUNK)], idx_v)
            pltpu.sync_copy(w16_hbm.at[pl.ds(base, CHUNK)], wbuf)
            lanes = lax.iota(jnp.int32, 16)
            for g in range(CHUNK // 16):
                ebase = idx_v[pl.ds(g * 16, 16)] * H
                erow = lanes + (g * 16)
                for h in range(H):
                    hvec = jnp.full((16,), h, jnp.int32)
                    vals = plsc.load_gather(wbuf, [erow, hvec])
                    plsc.addupdate_scatter(stab, [ebase + h], vals)

        return carry

    lax.fori_loop(0, ITERS, body, 0)
    pltpu.sync_copy(stab, s_all_hbm.at[cid, sid])


def _final_kernel(num_ref, s_all_ref, s_skip_ref, selt_ref, w1_ref, b1_ref,
                  w2_ref, b2_ref, g1_ref, be1_ref, g2_ref, be2_ref, out_ref):
    num = num_ref[0] + num_ref[1]
    s8 = jnp.sum(s_all_ref[0, 0], axis=0) + jnp.sum(s_all_ref[0, 1], axis=0)
    sfull = jnp.dot(s8, selt_ref[...], preferred_element_type=jnp.float32) + 1e-16
    h = num / sfull + s_skip_ref[...]
    mu = jnp.mean(h, axis=-1, keepdims=True)
    var = jnp.mean((h - mu) ** 2, axis=-1, keepdims=True)
    h = (h - mu) * lax.rsqrt(var + 1e-5) * g1_ref[...] + be1_ref[...]
    ff = jnp.maximum(
        jnp.dot(h, w1_ref[...], preferred_element_type=jnp.float32) + b1_ref[...], 0.0)
    ff = jnp.dot(ff, w2_ref[...], preferred_element_type=jnp.float32) + b2_ref[...]
    h = h + ff
    mu = jnp.mean(h, axis=-1, keepdims=True)
    var = jnp.mean((h - mu) ** 2, axis=-1, keepdims=True)
    out_ref[...] = (h - mu) * lax.rsqrt(var + 1e-5) * g2_ref[...] + be2_ref[...]


def kernel(x, edge_index, edge_attr, Wq, bq, Wk, bk, Wv, bv, Ws, bs,
           g1, be1, W1, b1, W2, b2, g2, be2):
    src = edge_index[0]
    dst = edge_index[1]
    scale = 1.0 / math.sqrt(C)

    wall = jnp.concatenate([Wq * scale, Wk, Wv, Ws], axis=1)
    ball = jnp.concatenate([bq * scale, bk, bv, bs]).reshape(1, 4 * D)
    wkv = jnp.concatenate([Wk, Wv], axis=1)
    sel = (jnp.arange(D)[:, None] // C == jnp.arange(H)[None, :]).astype(jnp.float32)
    selt = sel.T

    q_t, kv_t, s_t = pl.pallas_call(
        _node_tables_kernel,
        grid=(N // RN,),
        in_specs=[pl.BlockSpec((RN, D), lambda i: (i, 0)),
                  pl.BlockSpec((D, 4 * D), lambda i: (0, 0)),
                  pl.BlockSpec((1, 4 * D), lambda i: (0, 0))],
        out_specs=[pl.BlockSpec((RN, D), lambda i: (i, 0)),
                   pl.BlockSpec((RN, DK), lambda i: (i, 0)),
                   pl.BlockSpec((RN, D), lambda i: (i, 0))],
        out_shape=[jax.ShapeDtypeStruct((N, D), jnp.float32),
                   jax.ShapeDtypeStruct((N, DK), jnp.bfloat16),
                   jax.ShapeDtypeStruct((N, D), jnp.float32)],
    )(x, wall, ball)

    mesh = plsc.VectorSubcoreMesh(core_axis_name="c", subcore_axis_name="s")
    gq, gkv = pl.kernel(
        _gather_body,
        out_type=[jax.ShapeDtypeStruct((E, D), jnp.float32),
                  jax.ShapeDtypeStruct((E, DK // 2), jnp.float32)],
        mesh=mesh,
        scratch_types=[pltpu.VMEM((CHUNK,), jnp.int32),
                       pltpu.VMEM((CHUNK,), jnp.int32),
                       pltpu.VMEM((CHUNK, D), jnp.float32),
                       pltpu.VMEM((CHUNK, DK // 2), jnp.float32),
                       pltpu.SemaphoreType.DMA,
                       pltpu.SemaphoreType.DMA],
    )(q_t, kv_t.view(jnp.float32), src, dst)
    gkv = gkv.view(jnp.bfloat16)

    wv, w16 = pl.pallas_call(
        _edge_kernel,
        grid=(E // BE,),
        in_specs=[pl.BlockSpec((BE, D), lambda i: (i, 0)),
                  pl.BlockSpec((BE, D), lambda i: (i, 0)),
                  pl.BlockSpec((BE, DK), lambda i: (i, 0)),
                  pl.BlockSpec((D, DK), lambda i: (0, 0)),
                  pl.BlockSpec((D, H), lambda i: (0, 0)),
                  pl.BlockSpec((H, D), lambda i: (0, 0))],
        out_specs=[pl.BlockSpec((BE, D), lambda i: (i, 0)),
                   pl.BlockSpec((BE, WPAD), lambda i: (i, 0))],
        out_shape=[jax.ShapeDtypeStruct((E, D), jnp.float32),
                   jax.ShapeDtypeStruct((E, WPAD), jnp.float32)],
    )(edge_attr, gq, gkv, wkv, sel, selt)

    zn = jnp.zeros((NP, D), jnp.float32)
    zt = jnp.zeros((NP * H,), jnp.float32)
    nump = pl.kernel(
        _scatter_body,
        out_type=jax.ShapeDtypeStruct((NC, NP, D), jnp.float32),
        mesh=mesh,
        scratch_types=[pltpu.VMEM((CHUNK,), jnp.int32),
                       pltpu.VMEM((CHUNK, D), jnp.float32),
                       pltpu.VMEM_SHARED((NP, D), jnp.float32),
                       pltpu.SemaphoreType.DMA],
    )(wv, dst, zn)
    s_all = pl.kernel(
        _sums_body,
        out_type=jax.ShapeDtypeStruct((NC, NS, NP * H), jnp.float32),
        mesh=mesh,
        compiler_params=pltpu.CompilerParams(needs_layout_passes=False),
        scratch_types=[pltpu.VMEM((CHUNK,), jnp.int32),
                       pltpu.VMEM((CHUNK, WPAD), jnp.float32),
                       pltpu.VMEM((NP * H,), jnp.float32)],
    )(w16, dst, zt)
    s_all = s_all.reshape(NC, NS, NP, H)

    out = pl.pallas_call(
        _final_kernel,
        grid=(NP // RN2,),
        in_specs=[pl.BlockSpec((NC, RN2, D), lambda i: (0, i, 0)),
                  pl.BlockSpec((1, NC, NS, RN2, H), lambda i: (0, 0, 0, i, 0)),
                  pl.BlockSpec((RN2, D), lambda i: (i, 0)),
                  pl.BlockSpec((H, D), lambda i: (0, 0)),
                  pl.BlockSpec((D, 2 * D), lambda i: (0, 0)),
                  pl.BlockSpec((1, 2 * D), lambda i: (0, 0)),
                  pl.BlockSpec((2 * D, D), lambda i: (0, 0)),
                  pl.BlockSpec((1, D), lambda i: (0, 0)),
                  pl.BlockSpec((1, D), lambda i: (0, 0)),
                  pl.BlockSpec((1, D), lambda i: (0, 0)),
                  pl.BlockSpec((1, D), lambda i: (0, 0)),
                  pl.BlockSpec((1, D), lambda i: (0, 0))],
        out_specs=pl.BlockSpec((RN2, D), lambda i: (i, 0)),
        out_shape=jax.ShapeDtypeStruct((N, D), jnp.float32),
    )(nump, s_all.reshape(1, NC, NS, NP, H), s_t, selt, W1, b1.reshape(1, 2 * D), W2, b2.reshape(1, D),
      g1.reshape(1, D), be1.reshape(1, D), g2.reshape(1, D), be2.reshape(1, D))

    return out


# speedup vs baseline: 1.8869x; 1.8869x over previous
import math

import jax
import jax.numpy as jnp
from jax import lax
from jax.experimental import pallas as pl
from jax.experimental.pallas import tpu as pltpu
from jax.experimental.pallas import tpu_sc as plsc

N = 10000
E = 320000
D = 128
H = 8
C = D // H
DK = 2 * D
WPAD = 16

NC, NS = 2, 16
NW = NC * NS
CHUNK = 128
NCHUNK = E // CHUNK
ITERS = (NCHUNK + NW - 1) // NW
NP = 10240
NHALF = NP // NC
SROWS = NHALF + 8
RPT_HALF = NHALF // NS
ITERS2 = (NCHUNK + NS - 1) // NS

RN = 1000
RN2 = 640
BE = 2000


def _node_tables_kernel(x_ref, wall_ref, ball_ref, q_ref, kv_ref, s_ref):
    t = jnp.dot(x_ref[...], wall_ref[...], preferred_element_type=jnp.float32) + ball_ref[...]
    q_ref[...] = t[:, :D]
    kv_ref[...] = t[:, D:D + DK]
    s_ref[...] = x_ref[...] + t[:, D + DK:]


def _gather_body(q_hbm, kv_hbm, src_hbm, dst_hbm, gq_hbm, gkv_hbm,
                 idxs_v, idxd_v, qbuf, kvbuf, sem1, sem2):
    cid = lax.axis_index("c")
    sid = lax.axis_index("s")
    wid = sid * NC + cid

    def body(j, carry):
        c = j * NW + wid

        @pl.when(c < NCHUNK)
        def _():
            base = c * CHUNK
            pltpu.sync_copy(src_hbm.at[pl.ds(base, CHUNK)], idxs_v)
            pltpu.sync_copy(dst_hbm.at[pl.ds(base, CHUNK)], idxd_v)
            cp1 = pltpu.async_copy(kv_hbm.at[idxs_v], kvbuf, sem1)
            cp2 = pltpu.async_copy(q_hbm.at[idxd_v], qbuf, sem2)
            cp1.wait()
            cp2.wait()
            pltpu.sync_copy(kvbuf, gkv_hbm.at[pl.ds(base, CHUNK)])
            pltpu.sync_copy(qbuf, gq_hbm.at[pl.ds(base, CHUNK)])

        return carry

    lax.fori_loop(0, ITERS, body, 0)


def _edge_kernel(ea_ref, gq_ref, gkv_ref, wkv_ref, sel_ref, selt_ref,
                 wv_ref, w16_ref):
    ekv = jnp.dot(ea_ref[...], wkv_ref[...], preferred_element_type=jnp.float32)
    k = gkv_ref[:, :D] + ekv[:, :D]
    v = gkv_ref[:, D:] + ekv[:, D:]
    alpha = jnp.dot(gq_ref[...] * k, sel_ref[...],
                    preferred_element_type=jnp.float32)
    w = jnp.exp(alpha)
    wfull = jnp.dot(w, selt_ref[...], preferred_element_type=jnp.float32)
    wv_ref[...] = v * wfull
    w16_ref[...] = jnp.concatenate([w, jnp.zeros_like(w)], axis=1)


def _scatter_body(wv_hbm, dst_hbm, zn_hbm, nump_hbm,
                  idx_v, wvbuf, num_sh, sem):
    cid = lax.axis_index("c")
    sid = lax.axis_index("s")
    wid = sid * NC + cid
    r0 = sid * (NP // NS)

    pltpu.sync_copy(zn_hbm.at[pl.ds(r0, NP // NS)],
                    num_sh.at[pl.ds(r0, NP // NS)])
    plsc.subcore_barrier()

    def body(j, carry):
        c = j * NW + wid

        @pl.when(c < NCHUNK)
        def _():
            base = c * CHUNK
            pltpu.sync_copy(dst_hbm.at[pl.ds(base, CHUNK)], idx_v)
            pltpu.sync_copy(wv_hbm.at[pl.ds(base, CHUNK)], wvbuf)
            pltpu.sync_copy(wvbuf, num_sh.at[idx_v], add=True)

        return carry

    lax.fori_loop(0, ITERS, body, 0)
    plsc.subcore_barrier()

    pltpu.sync_copy(num_sh.at[pl.ds(r0, NP // NS)],
                    nump_hbm.at[cid, pl.ds(r0, NP // NS)])


def _sums_body(w16_hbm, dst_hbm, zt_hbm, s_all_hbm, idx_v, wbuf, stab):
    cid = lax.axis_index("c")
    sid = lax.axis_index("s")
    wid = sid * NC + cid

    pltpu.sync_copy(zt_hbm, stab)

    def body(j, carry):
        c = j * NW + wid

        @pl.when(c < NCHUNK)
        def _():
            base = c * CHUNK
            pltpu.sync_copy(dst_hbm.at[pl.ds(base, CHUNK)], idx_v)
            pltpu.sync_copy(w16_hbm.at[pl.ds(base, CHUNK)], wbuf)
            lanes = lax.iota(jnp.int32, 16)
            for g in range(CHUNK // 16):
                ebase = idx_v[pl.ds(g * 16, 16)] * H
                erow = lanes + (g * 16)
                for h in range(H):
                    hvec = jnp.full((16,), h, jnp.int32)
                    vals = plsc.load_gather(wbuf, [erow, hvec])
                    plsc.addupdate_scatter(stab, [ebase + h], vals)

        return carry

    lax.fori_loop(0, ITERS, body, 0)
    pltpu.sync_copy(stab, s_all_hbm.at[cid, sid])


def _final_kernel(num_ref, s_all_ref, s_skip_ref, selt_ref, w1_ref, b1_ref,
                  w2_ref, b2_ref, g1_ref, be1_ref, g2_ref, be2_ref, out_ref):
    num = num_ref[0] + num_ref[1]
    s8 = jnp.sum(s_all_ref[0, 0], axis=0) + jnp.sum(s_all_ref[0, 1], axis=0)
    sfull = jnp.dot(s8, selt_ref[...], preferred_element_type=jnp.float32) + 1e-16
    h = num / sfull + s_skip_ref[...]
    mu = jnp.mean(h, axis=-1, keepdims=True)
    var = jnp.mean((h - mu) ** 2, axis=-1, keepdims=True)
    h = (h - mu) * lax.rsqrt(var + 1e-5) * g1_ref[...] + be1_ref[...]
    ff = jnp.maximum(
        jnp.dot(h, w1_ref[...], preferred_element_type=jnp.float32) + b1_ref[...], 0.0)
    ff = jnp.dot(ff, w2_ref[...], preferred_element_type=jnp.float32) + b2_ref[...]
    h = h + ff
    mu = jnp.mean(h, axis=-1, keepdims=True)
    var = jnp.mean((h - mu) ** 2, axis=-1, keepdims=True)
    out_ref[...] = (h - mu) * lax.rsqrt(var + 1e-5) * g2_ref[...] + be2_ref[...]


def kernel(x, edge_index, edge_attr, Wq, bq, Wk, bk, Wv, bv, Ws, bs,
           g1, be1, W1, b1, W2, b2, g2, be2):
    src = edge_index[0]
    dst = edge_index[1]
    scale = 1.0 / math.sqrt(C)

    wall = jnp.concatenate([Wq * scale, Wk, Wv, Ws], axis=1)
    ball = jnp.concatenate([bq * scale, bk, bv, bs]).reshape(1, 4 * D)
    wkv = jnp.concatenate([Wk, Wv], axis=1)
    sel = (jnp.arange(D)[:, None] // C == jnp.arange(H)[None, :]).astype(jnp.float32)
    selt = sel.T

    q_t, kv_t, s_t = pl.pallas_call(
        _node_tables_kernel,
        grid=(N // RN,),
        in_specs=[pl.BlockSpec((RN, D), lambda i: (i, 0)),
                  pl.BlockSpec((D, 4 * D), lambda i: (0, 0)),
                  pl.BlockSpec((1, 4 * D), lambda i: (0, 0))],
        out_specs=[pl.BlockSpec((RN, D), lambda i: (i, 0)),
                   pl.BlockSpec((RN, DK), lambda i: (i, 0)),
                   pl.BlockSpec((RN, D), lambda i: (i, 0))],
        out_shape=[jax.ShapeDtypeStruct((N, D), jnp.float32),
                   jax.ShapeDtypeStruct((N, DK), jnp.float32),
                   jax.ShapeDtypeStruct((N, D), jnp.float32)],
    )(x, wall, ball)

    mesh = plsc.VectorSubcoreMesh(core_axis_name="c", subcore_axis_name="s")
    gq, gkv = pl.kernel(
        _gather_body,
        out_type=[jax.ShapeDtypeStruct((E, D), jnp.float32),
                  jax.ShapeDtypeStruct((E, DK), jnp.float32)],
        mesh=mesh,
        scratch_types=[pltpu.VMEM((CHUNK,), jnp.int32),
                       pltpu.VMEM((CHUNK,), jnp.int32),
                       pltpu.VMEM((CHUNK, D), jnp.float32),
                       pltpu.VMEM((CHUNK, DK), jnp.float32),
                       pltpu.SemaphoreType.DMA,
                       pltpu.SemaphoreType.DMA],
    )(q_t, kv_t, src, dst)

    wv, w16 = pl.pallas_call(
        _edge_kernel,
        grid=(E // BE,),
        in_specs=[pl.BlockSpec((BE, D), lambda i: (i, 0)),
                  pl.BlockSpec((BE, D), lambda i: (i, 0)),
                  pl.BlockSpec((BE, DK), lambda i: (i, 0)),
                  pl.BlockSpec((D, DK), lambda i: (0, 0)),
                  pl.BlockSpec((D, H), lambda i: (0, 0)),
                  pl.BlockSpec((H, D), lambda i: (0, 0))],
        out_specs=[pl.BlockSpec((BE, D), lambda i: (i, 0)),
                   pl.BlockSpec((BE, WPAD), lambda i: (i, 0))],
        out_shape=[jax.ShapeDtypeStruct((E, D), jnp.float32),
                   jax.ShapeDtypeStruct((E, WPAD), jnp.float32)],
    )(edge_attr, gq, gkv, wkv, sel, selt)

    zn = jnp.zeros((NP, D), jnp.float32)
    zt = jnp.zeros((NP * H,), jnp.float32)
    nump = pl.kernel(
        _scatter_body,
        out_type=jax.ShapeDtypeStruct((NC, NP, D), jnp.float32),
        mesh=mesh,
        scratch_types=[pltpu.VMEM((CHUNK,), jnp.int32),
                       pltpu.VMEM((CHUNK, D), jnp.float32),
                       pltpu.VMEM_SHARED((NP, D), jnp.float32),
                       pltpu.SemaphoreType.DMA],
    )(wv, dst, zn)
    s_all = pl.kernel(
        _sums_body,
        out_type=jax.ShapeDtypeStruct((NC, NS, NP * H), jnp.float32),
        mesh=mesh,
        compiler_params=pltpu.CompilerParams(needs_layout_passes=False),
        scratch_types=[pltpu.VMEM((CHUNK,), jnp.int32),
                       pltpu.VMEM((CHUNK, WPAD), jnp.float32),
                       pltpu.VMEM((NP * H,), jnp.float32)],
    )(w16, dst, zt)
    s_all = s_all.reshape(NC, NS, NP, H)

    out = pl.pallas_call(
        _final_kernel,
        grid=(NP // RN2,),
        in_specs=[pl.BlockSpec((NC, RN2, D), lambda i: (0, i, 0)),
                  pl.BlockSpec((1, NC, NS, RN2, H), lambda i: (0, 0, 0, i, 0)),
                  pl.BlockSpec((RN2, D), lambda i: (i, 0)),
                  pl.BlockSpec((H, D), lambda i: (0, 0)),
                  pl.BlockSpec((D, 2 * D), lambda i: (0, 0)),
                  pl.BlockSpec((1, 2 * D), lambda i: (0, 0)),
                  pl.BlockSpec((2 * D, D), lambda i: (0, 0)),
                  pl.BlockSpec((1, D), lambda i: (0, 0)),
                  pl.BlockSpec((1, D), lambda i: (0, 0)),
                  pl.BlockSpec((1, D), lambda i: (0, 0)),
                  pl.BlockSpec((1, D), lambda i: (0, 0)),
                  pl.BlockSpec((1, D), lambda i: (0, 0))],
        out_specs=pl.BlockSpec((RN2, D), lambda i: (i, 0)),
        out_shape=jax.ShapeDtypeStruct((N, D), jnp.float32),
    )(nump, s_all.reshape(1, NC, NS, NP, H), s_t, selt, W1, b1.reshape(1, 2 * D), W2, b2.reshape(1, D),
      g1.reshape(1, D), be1.reshape(1, D), g2.reshape(1, D), be2.reshape(1, D))

    return out


# speedup vs baseline: 1.9422x; 1.0293x over previous
import math

import jax
import jax.numpy as jnp
from jax import lax
from jax.experimental import pallas as pl
from jax.experimental.pallas import tpu as pltpu
from jax.experimental.pallas import tpu_sc as plsc

N = 10000
E = 320000
D = 128
H = 8
C = D // H
DK = 2 * D
WPAD = 16

NC, NS = 2, 16
NW = NC * NS
CHUNK = 128
NCHUNK = E // CHUNK
ITERS = (NCHUNK + NW - 1) // NW
NP = 10240
NHALF = NP // NC
SROWS = NHALF + 8
RPT_HALF = NHALF // NS
ITERS2 = (NCHUNK + NS - 1) // NS

RN = 1000
RN2 = 640
BE = 2000


def _node_tables_kernel(x_ref, wall_ref, ball_ref, q_ref, kv_ref, s_ref):
    t = jnp.dot(x_ref[...], wall_ref[...], preferred_element_type=jnp.float32) + ball_ref[...]
    q_ref[...] = t[:, :D]
    kv_ref[...] = t[:, D:D + DK]
    s_ref[...] = x_ref[...] + t[:, D + DK:]


def _gather_body(q_hbm, kv_hbm, src_hbm, dst_hbm, gq_hbm, gkv_hbm,
                 idxs_v, idxd_v, qbuf, kvbuf, sem1, sem2):
    cid = lax.axis_index("c")
    sid = lax.axis_index("s")
    wid = sid * NC + cid

    def body(j, carry):
        c = j * NW + wid

        @pl.when(c < NCHUNK)
        def _():
            base = c * CHUNK
            ci1 = pltpu.async_copy(src_hbm.at[pl.ds(base, CHUNK)], idxs_v, sem1)
            ci2 = pltpu.async_copy(dst_hbm.at[pl.ds(base, CHUNK)], idxd_v, sem2)
            ci1.wait()
            ci2.wait()
            cp1 = pltpu.async_copy(kv_hbm.at[idxs_v], kvbuf, sem1)
            cp2 = pltpu.async_copy(q_hbm.at[idxd_v], qbuf, sem2)
            cp1.wait()
            cp2.wait()
            cw1 = pltpu.async_copy(kvbuf, gkv_hbm.at[pl.ds(base, CHUNK)], sem1)
            cw2 = pltpu.async_copy(qbuf, gq_hbm.at[pl.ds(base, CHUNK)], sem2)
            cw1.wait()
            cw2.wait()

        return carry

    lax.fori_loop(0, ITERS, body, 0)


def _edge_kernel(ea_ref, gq_ref, gkv_ref, wkv_ref, sel_ref, selt_ref,
                 wv_ref, w16_ref):
    ekv = jnp.dot(ea_ref[...], wkv_ref[...], preferred_element_type=jnp.float32)
    k = gkv_ref[:, :D] + ekv[:, :D]
    v = gkv_ref[:, D:] + ekv[:, D:]
    alpha = jnp.dot(gq_ref[...] * k, sel_ref[...],
                    preferred_element_type=jnp.float32)
    w = jnp.exp(alpha)
    wfull = jnp.dot(w, selt_ref[...], preferred_element_type=jnp.float32)
    wv_ref[...] = v * wfull
    w16_ref[...] = jnp.concatenate([w, jnp.zeros_like(w)], axis=1)


def _scatter_body(wv_hbm, dst_hbm, zn_hbm, nump_hbm,
                  idx_v, wvbuf, num_sh, sem):
    cid = lax.axis_index("c")
    sid = lax.axis_index("s")
    wid = sid * NC + cid
    r0 = sid * (NP // NS)

    pltpu.sync_copy(zn_hbm.at[pl.ds(r0, NP // NS)],
                    num_sh.at[pl.ds(r0, NP // NS)])
    plsc.subcore_barrier()

    def body(j, carry):
        c = j * NW + wid

        @pl.when(c < NCHUNK)
        def _():
            base = c * CHUNK
            pltpu.sync_copy(dst_hbm.at[pl.ds(base, CHUNK)], idx_v)
            pltpu.sync_copy(wv_hbm.at[pl.ds(base, CHUNK)], wvbuf)
            pltpu.sync_copy(wvbuf, num_sh.at[idx_v], add=True)

        return carry

    lax.fori_loop(0, ITERS, body, 0)
    plsc.subcore_barrier()

    pltpu.sync_copy(num_sh.at[pl.ds(r0, NP // NS)],
                    nump_hbm.at[cid, pl.ds(r0, NP // NS)])


def _sums_body(w16_hbm, dst_hbm, zt_hbm, s_all_hbm, idx_v, wbuf, stab):
    cid = lax.axis_index("c")
    sid = lax.axis_index("s")
    wid = sid * NC + cid

    pltpu.sync_copy(zt_hbm, stab)

    def body(j, carry):
        c = j * NW + wid

        @pl.when(c < NCHUNK)
        def _():
            base = c * CHUNK
            pltpu.sync_copy(dst_hbm.at[pl.ds(base, CHUNK)], idx_v)
            pltpu.sync_copy(w16_hbm.at[pl.ds(base, CHUNK)], wbuf)
            lanes = lax.iota(jnp.int32, 16)
            for g in range(CHUNK // 16):
                ebase = idx_v[pl.ds(g * 16, 16)] * H
                erow = lanes + (g * 16)
                for h in range(H):
                    hvec = jnp.full((16,), h, jnp.int32)
                    vals = plsc.load_gather(wbuf, [erow, hvec])
                    plsc.addupdate_scatter(stab, [ebase + h], vals)

        return carry

    lax.fori_loop(0, ITERS, body, 0)
    pltpu.sync_copy(stab, s_all_hbm.at[cid, sid])


def _final_kernel(num_ref, s_all_ref, s_skip_ref, selt_ref, w1_ref, b1_ref,
                  w2_ref, b2_ref, g1_ref, be1_ref, g2_ref, be2_ref, out_ref):
    num = num_ref[0] + num_ref[1]
    s8 = jnp.sum(s_all_ref[0, 0], axis=0) + jnp.sum(s_all_ref[0, 1], axis=0)
    sfull = jnp.dot(s8, selt_ref[...], preferred_element_type=jnp.float32) + 1e-16
    h = num / sfull + s_skip_ref[...]
    mu = jnp.mean(h, axis=-1, keepdims=True)
    var = jnp.mean((h - mu) ** 2, axis=-1, keepdims=True)
    h = (h - mu) * lax.rsqrt(var + 1e-5) * g1_ref[...] + be1_ref[...]
    ff = jnp.maximum(
        jnp.dot(h, w1_ref[...], preferred_element_type=jnp.float32) + b1_ref[...], 0.0)
    ff = jnp.dot(ff, w2_ref[...], preferred_element_type=jnp.float32) + b2_ref[...]
    h = h + ff
    mu = jnp.mean(h, axis=-1, keepdims=True)
    var = jnp.mean((h - mu) ** 2, axis=-1, keepdims=True)
    out_ref[...] = (h - mu) * lax.rsqrt(var + 1e-5) * g2_ref[...] + be2_ref[...]


def kernel(x, edge_index, edge_attr, Wq, bq, Wk, bk, Wv, bv, Ws, bs,
           g1, be1, W1, b1, W2, b2, g2, be2):
    src = edge_index[0]
    dst = edge_index[1]
    scale = 1.0 / math.sqrt(C)

    wall = jnp.concatenate([Wq * scale, Wk, Wv, Ws], axis=1)
    ball = jnp.concatenate([bq * scale, bk, bv, bs]).reshape(1, 4 * D)
    wkv = jnp.concatenate([Wk, Wv], axis=1)
    sel = (jnp.arange(D)[:, None] // C == jnp.arange(H)[None, :]).astype(jnp.float32)
    selt = sel.T

    q_t, kv_t, s_t = pl.pallas_call(
        _node_tables_kernel,
        grid=(N // RN,),
        in_specs=[pl.BlockSpec((RN, D), lambda i: (i, 0)),
                  pl.BlockSpec((D, 4 * D), lambda i: (0, 0)),
                  pl.BlockSpec((1, 4 * D), lambda i: (0, 0))],
        out_specs=[pl.BlockSpec((RN, D), lambda i: (i, 0)),
                   pl.BlockSpec((RN, DK), lambda i: (i, 0)),
                   pl.BlockSpec((RN, D), lambda i: (i, 0))],
        out_shape=[jax.ShapeDtypeStruct((N, D), jnp.float32),
                   jax.ShapeDtypeStruct((N, DK), jnp.float32),
                   jax.ShapeDtypeStruct((N, D), jnp.float32)],
    )(x, wall, ball)

    mesh = plsc.VectorSubcoreMesh(core_axis_name="c", subcore_axis_name="s")
    gq, gkv = pl.kernel(
        _gather_body,
        out_type=[jax.ShapeDtypeStruct((E, D), jnp.float32),
                  jax.ShapeDtypeStruct((E, DK), jnp.float32)],
        mesh=mesh,
        scratch_types=[pltpu.VMEM((CHUNK,), jnp.int32),
                       pltpu.VMEM((CHUNK,), jnp.int32),
                       pltpu.VMEM((CHUNK, D), jnp.float32),
                       pltpu.VMEM((CHUNK, DK), jnp.float32),
                       pltpu.SemaphoreType.DMA,
                       pltpu.SemaphoreType.DMA],
    )(q_t, kv_t, src, dst)

    wv, w16 = pl.pallas_call(
        _edge_kernel,
        grid=(E // BE,),
        in_specs=[pl.BlockSpec((BE, D), lambda i: (i, 0)),
                  pl.BlockSpec((BE, D), lambda i: (i, 0)),
                  pl.BlockSpec((BE, DK), lambda i: (i, 0)),
                  pl.BlockSpec((D, DK), lambda i: (0, 0)),
                  pl.BlockSpec((D, H), lambda i: (0, 0)),
                  pl.BlockSpec((H, D), lambda i: (0, 0))],
        out_specs=[pl.BlockSpec((BE, D), lambda i: (i, 0)),
                   pl.BlockSpec((BE, WPAD), lambda i: (i, 0))],
        out_shape=[jax.ShapeDtypeStruct((E, D), jnp.float32),
                   jax.ShapeDtypeStruct((E, WPAD), jnp.float32)],
    )(edge_attr, gq, gkv, wkv, sel, selt)

    zn = jnp.zeros((NP, D), jnp.float32)
    zt = jnp.zeros((NP * H,), jnp.float32)
    nump = pl.kernel(
        _scatter_body,
        out_type=jax.ShapeDtypeStruct((NC, NP, D), jnp.float32),
        mesh=mesh,
        scratch_types=[pltpu.VMEM((CHUNK,), jnp.int32),
                       pltpu.VMEM((CHUNK, D), jnp.float32),
                       pltpu.VMEM_SHARED((NP, D), jnp.float32),
                       pltpu.SemaphoreType.DMA],
    )(wv, dst, zn)
    s_all = pl.kernel(
        _sums_body,
        out_type=jax.ShapeDtypeStruct((NC, NS, NP * H), jnp.float32),
        mesh=mesh,
        compiler_params=pltpu.CompilerParams(needs_layout_passes=False),
        scratch_types=[pltpu.VMEM((CHUNK,), jnp.int32),
                       pltpu.VMEM((CHUNK, WPAD), jnp.float32),
                       pltpu.VMEM((NP * H,), jnp.float32)],
    )(w16, dst, zt)
    s_all = s_all.reshape(NC, NS, NP, H)

    out = pl.pallas_call(
        _final_kernel,
        grid=(NP // RN2,),
        in_specs=[pl.BlockSpec((NC, RN2, D), lambda i: (0, i, 0)),
                  pl.BlockSpec((1, NC, NS, RN2, H), lambda i: (0, 0, 0, i, 0)),
                  pl.BlockSpec((RN2, D), lambda i: (i, 0)),
                  pl.BlockSpec((H, D), lambda i: (0, 0)),
                  pl.BlockSpec((D, 2 * D), lambda i: (0, 0)),
                  pl.BlockSpec((1, 2 * D), lambda i: (0, 0)),
                  pl.BlockSpec((2 * D, D), lambda i: (0, 0)),
                  pl.BlockSpec((1, D), lambda i: (0, 0)),
                  pl.BlockSpec((1, D), lambda i: (0, 0)),
                  pl.BlockSpec((1, D), lambda i: (0, 0)),
                  pl.BlockSpec((1, D), lambda i: (0, 0)),
                  pl.BlockSpec((1, D), lambda i: (0, 0))],
        out_specs=pl.BlockSpec((RN2, D), lambda i: (i, 0)),
        out_shape=jax.ShapeDtypeStruct((N, D), jnp.float32),
    )(nump, s_all.reshape(1, NC, NS, NP, H), s_t, selt, W1, b1.reshape(1, 2 * D), W2, b2.reshape(1, D),
      g1.reshape(1, D), be1.reshape(1, D), g2.reshape(1, D), be2.reshape(1, D))

    return out
